# baseline (speedup 1.0000x reference)
import functools

import jax
import jax.numpy as jnp
from jax import lax
from jax.experimental import pallas as pl
from jax.experimental.pallas import tpu as pltpu
from jax.experimental.pallas import tpu_sc as plsc

_L = 16
_NC = 2
_NS = 16
_NW = _NC * _NS
_C = 80
_D = 5


def _precompute_tables(z, W1, b1):
    n, d = z.shape
    h = W1.shape[0]

    def body(z_ref, w1_ref, b1_ref, za_ref, zb_ref):
        zz = z_ref[...]
        w1 = w1_ref[...]
        za = lax.dot_general(zz, w1[:, :d], (((1,), (1,)), ((), ())),
                             preferred_element_type=jnp.float32)
        zb = lax.dot_general(zz, w1[:, d:], (((1,), (1,)), ((), ())),
                             preferred_element_type=jnp.float32)
        za_ref[...] = (za + b1_ref[...]).astype(jnp.bfloat16)
        zb_ref[...] = zb.astype(jnp.bfloat16)

    return pl.pallas_call(
        body,
        out_shape=(jax.ShapeDtypeStruct((n, h), jnp.bfloat16),
                   jax.ShapeDtypeStruct((n, h), jnp.bfloat16)),
    )(z, W1, b1.reshape(1, h))


@functools.lru_cache(maxsize=None)
def _make_sc_scorer(n_edges, hid):
    epw = n_edges // _NW
    nch = epw // _C
    npair = hid // 2

    mesh = plsc.VectorSubcoreMesh(core_axis_name="c", subcore_axis_name="s")

    @functools.partial(
        pl.kernel,
        out_type=jax.ShapeDtypeStruct((_NW, nch, _C), jnp.float32),
        mesh=mesh,
        compiler_params=pltpu.CompilerParams(
            needs_layout_passes=False, use_tc_tiling_on_sc=False),
        scratch_types=[
            pltpu.VMEM((nch, _C), jnp.int32),
            pltpu.VMEM((nch, _C), jnp.int32),
            pltpu.VMEM((_D, _C, npair), jnp.int32),
            pltpu.VMEM((_D, _C, npair), jnp.int32),
            pltpu.VMEM((nch, _C), jnp.float32),
            pltpu.VMEM((npair + 1, _L), jnp.int32),
        ] + [pltpu.SemaphoreType.DMA] * (2 * _D),
    )
    def scorer(eidx_hbm, za_hbm, zb_hbm, wv_hbm, out_hbm,
               src_v, dst_v, rows_a, rows_b, out_v, wv_v,
               *sems):
        wid = lax.axis_index("s") * _NC + lax.axis_index("c")
        pltpu.sync_copy(eidx_hbm.at[0, wid], src_v)
        pltpu.sync_copy(eidx_hbm.at[1, wid], dst_v)
        pltpu.sync_copy(wv_hbm, wv_v)

        sems_a = sems[:_D]
        sems_b = sems[_D:]

        def gather_start(g, slot):
            pltpu.async_copy(za_hbm.at[src_v.at[g]], rows_a.at[slot], sems_a[slot])
            pltpu.async_copy(zb_hbm.at[dst_v.at[g]], rows_b.at[slot], sems_b[slot])

        def gather_wait(g, slot):
            pltpu.make_async_copy(
                za_hbm.at[src_v.at[g]], rows_a.at[slot], sems_a[slot]).wait()
            pltpu.make_async_copy(
                zb_hbm.at[dst_v.at[g]], rows_b.at[slot], sems_b[slot]).wait()

        wpk = [plsc.bitcast(wv_v[j], jnp.bfloat16) for j in range(npair)]
        b2v = plsc.bitcast(wv_v[npair], jnp.float32)
        zero = jnp.zeros((_L,), jnp.float32)
        one = jnp.ones((_L,), jnp.float32)
        zero_b = jnp.zeros((2 * _L,), jnp.bfloat16)
        lane = lax.iota(jnp.int32, _L)
        rowidx = [lane + (blk * _L) for blk in range(_C // _L)]
        wordidx = [jnp.full((_L,), j, jnp.int32) for j in range(npair)]
        slotidx = [jnp.full((_L,), s, jnp.int32) for s in range(_D)]

        def compute(g, slot):
            sv = slotidx[slot]
            for blk in range(_C // _L):
                rv = rowidx[blk]
                acc = [zero, zero, zero, zero]
                for j in range(npair):
                    ga = plsc.load_gather(rows_a, [sv, rv, wordidx[j]])
                    gb = plsc.load_gather(rows_b, [sv, rv, wordidx[j]])
                    t = jnp.maximum(
                        plsc.bitcast(ga, jnp.bfloat16)
                        + plsc.bitcast(gb, jnp.bfloat16), zero_b)
                    te, to = plsc.unpack(
                        t * wpk[j], format=plsc.PackFormat.INTERLEAVED,
                        preferred_element_type=jnp.float32)
                    k = 2 * (j % 2)
                    acc[k] = acc[k] + te
                    acc[k + 1] = acc[k + 1] + to
                x = (acc[0] + acc[1]) + (acc[2] + acc[3]) + b2v
                out_v[g, pl.ds(blk * _L, _L)] = one / (one + jnp.exp(-x))

        for s in range(_D - 1):
            gather_start(s, s)

        def ring_body(i, carry):
            for j in range(_D):
                g = _D * i + j
                gather_start(g + _D - 1, (j + _D - 1) % _D)
                gather_wait(g, j)
                compute(g, j)
            return carry

        lax.fori_loop(0, nch // _D - 1, ring_body, 0)
        base = nch - _D
        for j in range(_D):
            g = base + j
            if j < 1:
                gather_start(g + _D - 1, (j + _D - 1) % _D)
            gather_wait(g, j)
            compute(g, j)

        pltpu.sync_copy(out_v, out_hbm.at[wid])

    return scorer


def kernel(z, edge_index, W1, b1, W2, b2):
    n_edges = edge_index.shape[1]
    hid = W1.shape[0]
    n = z.shape[0]
    za, zb = _precompute_tables(z, W1, b1)
    za_p = lax.bitcast_convert_type(za.reshape(n, hid // 2, 2), jnp.int32)
    zb_p = lax.bitcast_convert_type(zb.reshape(n, hid // 2, 2), jnp.int32)
    eidx = edge_index.astype(jnp.int32).reshape(2, _NW, n_edges // (_NW * _C), _C)
    w2p = lax.bitcast_convert_type(
        W2.reshape(hid // 2, 2).astype(jnp.bfloat16), jnp.int32)
    wv = jnp.concatenate([
        jnp.broadcast_to(w2p[:, None], (hid // 2, _L)),
        lax.bitcast_convert_type(
            jnp.full((1, _L), b2[0], jnp.float32), jnp.int32),
    ]).astype(jnp.int32)
    out = _make_sc_scorer(n_edges, hid)(eidx, za_p, zb_p, wv)
    return out.reshape(-1)
